# Initial kernel scaffold; baseline (speedup 1.0000x reference)
import jax
import jax.numpy as jnp
from jax.experimental import pallas as pl


def kernel(center, outside, negative, emb_v, emb_u):
    raise NotImplementedError("write your pallas kernel here")


# speedup vs baseline: 1.0705x; 1.0705x over previous
import functools

import jax
import jax.numpy as jnp
from jax import lax
from jax.experimental import pallas as pl
from jax.experimental.pallas import tpu as pltpu
from jax.experimental.pallas import tpu_sc as plsc

B = 16384
N = 20
D = 32
NC = 2
NS = 16
NW = NC * NS
BPW = B // NW
C = 128
G = C // 16


def _log_sigmoid(x):
    z = jnp.exp(-jnp.abs(x))
    t = z / (z + 2.0)
    t2 = t * t
    p = 2.0 * t * (1.0 + t2 * (1.0 / 3.0 + t2 * (0.2 + t2 * (1.0 / 7.0 + t2 * (1.0 / 9.0)))))
    return jnp.minimum(x, 0.0) - p


@functools.partial(
    pl.kernel,
    mesh=plsc.VectorSubcoreMesh(core_axis_name="c", subcore_axis_name="s"),
    compiler_params=pltpu.CompilerParams(
        needs_layout_passes=False, use_tc_tiling_on_sc=False),
    out_type=jax.ShapeDtypeStruct((B,), jnp.float32),
    scratch_types=[
        pltpu.VMEM((C,), jnp.int32),
        pltpu.VMEM((C,), jnp.int32),
        pltpu.VMEM((C * N,), jnp.int32),
        pltpu.VMEM((C, D), jnp.float32),
        pltpu.VMEM((C, D), jnp.float32),
        pltpu.VMEM((C * N, D), jnp.float32),
        pltpu.VMEM((C,), jnp.float32),
        pltpu.SemaphoreType.DMA,
    ],
)
def _sc_loss(center_hbm, outside_hbm, negflat_hbm, emb_v_hbm, emb_u_hbm,
             out_hbm, cidx_v, oidx_v, nidx_v, vrows_v, orows_v, nrows_v,
             out_v, sem):
    wid = lax.axis_index("s") * NC + lax.axis_index("c")
    base = wid * BPW

    def chunk_body(c, carry):
        start = base + c * C
        pltpu.sync_copy(center_hbm.at[pl.ds(start, C)], cidx_v)
        pltpu.sync_copy(outside_hbm.at[pl.ds(start, C)], oidx_v)
        pltpu.sync_copy(negflat_hbm.at[pl.ds(start * N, C * N)], nidx_v)
        copies = [
            pltpu.async_copy(emb_v_hbm.at[cidx_v], vrows_v, sem),
            pltpu.async_copy(emb_u_hbm.at[oidx_v], orows_v, sem),
        ]
        for j in range(N):
            copies.append(pltpu.async_copy(
                emb_u_hbm.at[nidx_v.at[pl.ds(j * C, C)]],
                nrows_v.at[pl.ds(j * C, C)], sem))
        for cp in copies:
            cp.wait()

        def group_body(g, gcarry):
            zeros = jnp.zeros((16,), jnp.float32)
            lanes = lax.iota(jnp.int32, 16)

            def elem_body(j, accs):
                b = g * 16 + j
                mask = lanes == j
                v0 = vrows_v[b, pl.ds(0, 16)]
                v1 = vrows_v[b, pl.ds(16, 16)]
                o0 = orows_v[b, pl.ds(0, 16)]
                o1 = orows_v[b, pl.ds(16, 16)]
                tot = jnp.broadcast_to(jnp.sum(v0 * o0 + v1 * o1), (16,))
                new = [jnp.where(mask, tot, accs[0])]
                for n in range(N):
                    u0 = nrows_v[b * N + n, pl.ds(0, 16)]
                    u1 = nrows_v[b * N + n, pl.ds(16, 16)]
                    tot = jnp.broadcast_to(jnp.sum(v0 * u0 + v1 * u1), (16,))
                    new.append(jnp.where(mask, tot, accs[1 + n]))
                return tuple(new)

            accs = lax.fori_loop(0, 16, elem_body, (zeros,) * (N + 1))
            loss = _log_sigmoid(jnp.clip(accs[0], -10.0, 10.0))
            for n in range(N):
                loss = loss + _log_sigmoid(-jnp.clip(accs[1 + n], -10.0, 10.0))
            out_v[pl.ds(g * 16, 16)] = -loss
            return gcarry

        lax.fori_loop(0, G, group_body, 0)
        pltpu.sync_copy(out_v, out_hbm.at[pl.ds(start, C)])
        return carry

    lax.fori_loop(0, BPW // C, chunk_body, 0)


def kernel(center, outside, negative, emb_v, emb_u):
    return _sc_loss(center, outside, negative.reshape(-1), emb_v, emb_u)
